# Initial kernel scaffold; baseline (speedup 1.0000x reference)
import jax
import jax.numpy as jnp
from jax.experimental import pallas as pl


def kernel(node_feats, neighbor_feats, weight_1, weight_2, node_count):
    raise NotImplementedError("write your pallas kernel here")


# speedup vs baseline: 1.2133x; 1.2133x over previous
import jax
import jax.numpy as jnp
from jax.experimental import pallas as pl

_BLOCK = 512


def _body(w_ref, nf_ref, nb_ref, out_ref):
    nb = nb_ref[...]
    mean = jnp.sum(nb, axis=1) * (1.0 / nb.shape[1])
    x = jnp.concatenate([nf_ref[...], mean], axis=1)
    y = jax.lax.dot_general(
        w_ref[...], x, (((1,), (1,)), ((), ())),
        preferred_element_type=jnp.float32)
    out_ref[...] = jnp.maximum(y, 0.0)


def kernel(node_feats, neighbor_feats, weight_1, weight_2, node_count):
    n, f = node_feats.shape
    s = neighbor_feats.shape[0] // n
    e = weight_1.shape[0]
    scale = jnp.float32(node_count) / jnp.float32(n)
    z = jnp.zeros((e, f), jnp.float32)
    w = jnp.concatenate(
        [jnp.concatenate([weight_1, z], axis=1),
         jnp.concatenate([z, weight_2], axis=1)], axis=0) * scale
    nb3 = neighbor_feats.reshape(n, s, f)
    b = _BLOCK
    grid = (n + b - 1) // b
    return pl.pallas_call(
        _body,
        grid=(grid,),
        in_specs=[
            pl.BlockSpec((2 * e, 2 * f), lambda i: (0, 0)),
            pl.BlockSpec((b, f), lambda i: (i, 0)),
            pl.BlockSpec((b, s, f), lambda i: (i, 0, 0)),
        ],
        out_specs=pl.BlockSpec((2 * e, b), lambda i: (0, i)),
        out_shape=jax.ShapeDtypeStruct((2 * e, n), jnp.float32),
    )(w, node_feats, nb3)
